# Initial kernel scaffold; baseline (speedup 1.0000x reference)
import jax
import jax.numpy as jnp
from jax.experimental import pallas as pl


def kernel(x, edge_index, W_pre, b_pre, W1, b1, Wh, bh, Wo, bo):
    raise NotImplementedError("write your pallas kernel here")


# speedup vs baseline: 11.6583x; 11.6583x over previous
import functools

import jax
import jax.numpy as jnp
from jax import lax
from jax.experimental import pallas as pl
from jax.experimental.pallas import tpu as pltpu
from jax.experimental.pallas import tpu_sc as plsc

N = 10000
E = 320000
D_IN = 128
D_FEAT = 128
D_HID = 256
D_OUT = 128

N_PAD = 10240
K = 128
T = 158
E_PAD = 16 * T * K
RW = N_PAD // 16

_mesh = plsc.VectorSubcoreMesh(core_axis_name="c", subcore_axis_name="s")


@functools.partial(
    pl.kernel,
    out_type=(jax.ShapeDtypeStruct((N_PAD,), jnp.float32),
              jax.ShapeDtypeStruct((N_PAD,), jnp.float32)),
    mesh=_mesh,
    scratch_types=[
        pltpu.VMEM_SHARED((N_PAD,), jnp.float32),
        pltpu.VMEM((K,), jnp.int32),
        pltpu.VMEM((K,), jnp.int32),
        pltpu.VMEM((K,), jnp.float32),
        pltpu.VMEM((RW,), jnp.float32),
        pltpu.SemaphoreType.DMA,
        pltpu.SemaphoreType.DMA,
    ],
    name="gcn_degree",
)
def _deg_kernel(dst_r, ones_hbm, z_hbm, out_a, out_b,
                dacc, id0, id1, ones_v, buf, si0, si1):
    cid = lax.axis_index("c")
    sid = lax.axis_index("s")
    row0 = sid * RW
    my_dst = dst_r.at[sid]
    ids = (id0, id1)
    si = (si0, si1)
    pltpu.sync_copy(ones_hbm, ones_v)
    pltpu.sync_copy(z_hbm, buf)
    pltpu.sync_copy(buf, dacc.at[pl.ds(row0, RW)])
    plsc.subcore_barrier()
    th = T // 2

    def half(t0, nch, out_c):
        pltpu.async_copy(my_dst.at[t0], ids[0], si[0])
        pltpu.async_copy(my_dst.at[t0 + 1], ids[1], si[1])

        def step(l, b):
            pltpu.make_async_copy(my_dst.at[t0], ids[b], si[b]).wait()
            pltpu.sync_copy(ones_v, dacc.at[ids[b]], add=True)

            @pl.when(l + 2 < nch)
            def _():
                pltpu.async_copy(my_dst.at[t0 + l + 2], ids[b], si[b])

        def pairb(i, c):
            step(i * 2, 0)
            step(i * 2 + 1, 1)
            return c

        lax.fori_loop(0, nch // 2, pairb, None)
        if nch % 2:
            pltpu.make_async_copy(my_dst.at[t0], ids[0], si[0]).wait()
            pltpu.sync_copy(ones_v, dacc.at[ids[0]], add=True)
        plsc.subcore_barrier()
        sl = pl.ds(row0, RW)
        pltpu.sync_copy(dacc.at[sl], buf)
        pltpu.sync_copy(buf, out_c.at[sl])

    @pl.when(cid == 0)
    def _():
        half(0, th, out_a)

    @pl.when(cid == 1)
    def _():
        half(th, T - th, out_b)


_CONV_SCRATCH = [
    pltpu.VMEM_SHARED((N_PAD, 128), jnp.float32),
    pltpu.VMEM((K,), jnp.int32),
    pltpu.VMEM((K,), jnp.int32),
    pltpu.VMEM((K,), jnp.int32),
    pltpu.VMEM((K,), jnp.int32),
    pltpu.VMEM((K, 128), jnp.float32),
    pltpu.VMEM((K, 128), jnp.float32),
    pltpu.SemaphoreType.DMA,
    pltpu.SemaphoreType.DMA,
    pltpu.SemaphoreType.DMA,
    pltpu.SemaphoreType.DMA,
]


def _zero_acc(acc, row0, z_hbm, buf):
    pltpu.sync_copy(z_hbm, buf)
    for j in range(RW // K):
        pltpu.sync_copy(buf, acc.at[pl.ds(row0 + j * K, K)])
    plsc.subcore_barrier()


def _writeout(acc, row0, out_c, buf):
    plsc.subcore_barrier()
    for j in range(RW // K):
        sl = pl.ds(row0 + j * K, K)
        pltpu.sync_copy(acc.at[sl], buf)
        pltpu.sync_copy(buf, out_c.at[sl])


def _edge_loop(hs_c, acc, my_src, my_dst, t0, nchunks,
               iss, ids, rows, sg, si):
    def wait_idx(p):
        pltpu.make_async_copy(my_src.at[t0], iss[p], si[p]).wait()
        pltpu.make_async_copy(my_src.at[t0], ids[p], si[p]).wait()

    pltpu.async_copy(my_src.at[t0], iss[0], si[0])
    pltpu.async_copy(my_dst.at[t0], ids[0], si[0])
    pltpu.async_copy(my_src.at[t0 + 1], iss[1], si[1])
    pltpu.async_copy(my_dst.at[t0 + 1], ids[1], si[1])
    wait_idx(0)
    pltpu.async_copy(hs_c.at[iss[0]], rows[0], sg[0])

    def step(l, b):
        pltpu.make_async_copy(hs_c.at[pl.ds(0, K)], rows[b], sg[b]).wait()

        @pl.when(l + 1 < nchunks)
        def _():
            wait_idx(1 - b)
            pltpu.async_copy(hs_c.at[iss[1 - b]], rows[1 - b], sg[1 - b])

        pltpu.sync_copy(rows[b], acc.at[ids[b]], add=True)

        @pl.when(l + 2 < nchunks)
        def _():
            pltpu.async_copy(my_src.at[t0 + l + 2], iss[b], si[b])
            pltpu.async_copy(my_dst.at[t0 + l + 2], ids[b], si[b])

    def pair(i, c):
        step(i * 2, 0)
        step(i * 2 + 1, 1)
        return c

    lax.fori_loop(0, nchunks // 2, pair, None)
    if nchunks % 2:
        pltpu.make_async_copy(hs_c.at[pl.ds(0, K)], rows[0], sg[0]).wait()
        pltpu.sync_copy(rows[0], acc.at[ids[0]], add=True)


@functools.partial(
    pl.kernel,
    out_type=(jax.ShapeDtypeStruct((N_PAD, 128), jnp.float32),
              jax.ShapeDtypeStruct((N_PAD, 128), jnp.float32)),
    mesh=_mesh,
    scratch_types=_CONV_SCRATCH,
    name="gcn_conv_colsplit",
)
def _conv128(hs_a, hs_b, src_r, dst_r, z_hbm, out_a, out_b,
             acc, is0, is1, id0, id1, rows0, rows1, sg0, sg1, si0, si1):
    cid = lax.axis_index("c")
    sid = lax.axis_index("s")
    row0 = sid * RW
    my_src = src_r.at[sid]
    my_dst = dst_r.at[sid]
    bufs = ((is0, is1), (id0, id1), (rows0, rows1), (sg0, sg1), (si0, si1))
    _zero_acc(acc, row0, z_hbm, rows0)

    @pl.when(cid == 0)
    def _():
        _edge_loop(hs_a, acc, my_src, my_dst, 0, T, *bufs)
        _writeout(acc, row0, out_a, rows0)

    @pl.when(cid == 1)
    def _():
        _edge_loop(hs_b, acc, my_src, my_dst, 0, T, *bufs)
        _writeout(acc, row0, out_b, rows0)


@functools.partial(
    pl.kernel,
    out_type=(jax.ShapeDtypeStruct((N_PAD, 128), jnp.float32),
              jax.ShapeDtypeStruct((N_PAD, 128), jnp.float32)),
    mesh=_mesh,
    scratch_types=_CONV_SCRATCH,
    name="gcn_conv_edgesplit",
)
def _conv_es(hs, src_r, dst_r, z_hbm, out_a, out_b,
             acc, is0, is1, id0, id1, rows0, rows1, sg0, sg1, si0, si1):
    cid = lax.axis_index("c")
    sid = lax.axis_index("s")
    row0 = sid * RW
    my_src = src_r.at[sid]
    my_dst = dst_r.at[sid]
    bufs = ((is0, is1), (id0, id1), (rows0, rows1), (sg0, sg1), (si0, si1))
    _zero_acc(acc, row0, z_hbm, rows0)
    th = T // 2

    @pl.when(cid == 0)
    def _():
        _edge_loop(hs, acc, my_src, my_dst, 0, th, *bufs)
        _writeout(acc, row0, out_a, rows0)

    @pl.when(cid == 1)
    def _():
        _edge_loop(hs, acc, my_src, my_dst, th, T - th, *bufs)
        _writeout(acc, row0, out_b, rows0)


_R = 1024
_G = N_PAD // _R


def _dinv_of(da_ref, db_ref):
    return lax.rsqrt(da_ref[:] + db_ref[:] + 1.0)


def _mmA_body(x_ref, da_ref, db_ref, wp_ref, bp_ref, w1_ref, oa_ref, ob_ref):
    dinv = _dinv_of(da_ref, db_ref)
    t = jnp.dot(x_ref[:], wp_ref[:], preferred_element_type=jnp.float32, precision=lax.Precision.HIGHEST)
    t = t + bp_ref[0:1, :]
    hs = jnp.dot(t, w1_ref[:], preferred_element_type=jnp.float32, precision=lax.Precision.HIGHEST) * dinv
    oa_ref[:] = hs[:, :128]
    ob_ref[:] = hs[:, 128:]


def _mmA(x_pad, dega, degb, W_pre, b_pre8, W1):
    return pl.pallas_call(
        _mmA_body,
        grid=(_G,),
        in_specs=[
            pl.BlockSpec((_R, D_IN), lambda i: (i, 0)),
            pl.BlockSpec((_R, 1), lambda i: (i, 0)),
            pl.BlockSpec((_R, 1), lambda i: (i, 0)),
            pl.BlockSpec((D_IN, D_FEAT), lambda i: (0, 0)),
            pl.BlockSpec((8, D_FEAT), lambda i: (0, 0)),
            pl.BlockSpec((D_FEAT, D_HID), lambda i: (0, 0)),
        ],
        out_specs=[pl.BlockSpec((_R, 128), lambda i: (i, 0))] * 2,
        out_shape=[jax.ShapeDtypeStruct((N_PAD, 128), jnp.float32)] * 2,
    )(x_pad, dega, degb, W_pre, b_pre8, W1)


def _ep_g(aa, ab, ha, hb, da_ref, db_ref, b_ref, w_ref):
    dinv = _dinv_of(da_ref, db_ref)
    g0 = jnp.maximum((aa[:] + ha[:]) * dinv + b_ref[0:1, :128], 0.0)
    g1 = jnp.maximum((ab[:] + hb[:]) * dinv + b_ref[0:1, 128:], 0.0)
    h2 = (jnp.dot(g0, w_ref[:128, :], preferred_element_type=jnp.float32, precision=lax.Precision.HIGHEST)
          + jnp.dot(g1, w_ref[128:, :], preferred_element_type=jnp.float32, precision=lax.Precision.HIGHEST))
    return h2 * dinv


def _ep_body_split(aa, ab, ha, hb, da_ref, db_ref, b_ref, w_ref, oa, ob):
    h2 = _ep_g(aa, ab, ha, hb, da_ref, db_ref, b_ref, w_ref)
    half = h2.shape[1] // 2
    oa[:] = h2[:, :half]
    ob[:] = h2[:, half:]


def _ep_body_full(aa, ab, ha, hb, da_ref, db_ref, b_ref, w_ref, o):
    o[:] = _ep_g(aa, ab, ha, hb, da_ref, db_ref, b_ref, w_ref)


def _mm_ep(acc_a, acc_b, hs_a, hs_b, dega, degb, b8, W, d_out, split):
    if split:
        body = _ep_body_split
        out_specs = [pl.BlockSpec((_R, d_out // 2), lambda i: (i, 0))] * 2
        out_shape = [jax.ShapeDtypeStruct((N_PAD, d_out // 2),
                                          jnp.float32)] * 2
    else:
        body = _ep_body_full
        out_specs = pl.BlockSpec((_R, d_out), lambda i: (i, 0))
        out_shape = jax.ShapeDtypeStruct((N_PAD, d_out), jnp.float32)
    return pl.pallas_call(
        body,
        grid=(_G,),
        in_specs=[
            pl.BlockSpec((_R, 128), lambda i: (i, 0)),
            pl.BlockSpec((_R, 128), lambda i: (i, 0)),
            pl.BlockSpec((_R, 128), lambda i: (i, 0)),
            pl.BlockSpec((_R, 128), lambda i: (i, 0)),
            pl.BlockSpec((_R, 1), lambda i: (i, 0)),
            pl.BlockSpec((_R, 1), lambda i: (i, 0)),
            pl.BlockSpec((8, D_HID), lambda i: (0, 0)),
            pl.BlockSpec((D_HID, d_out), lambda i: (0, 0)),
        ],
        out_specs=out_specs,
        out_shape=out_shape,
    )(acc_a, acc_b, hs_a, hs_b, dega, degb, b8, W)


def _fin_body(p0, p1, h3, da_ref, db_ref, b_ref, o_ref):
    dinv = _dinv_of(da_ref, db_ref)
    h = (p0[:] + p1[:] + h3[:]) * dinv + b_ref[0:1, :]
    nrm = jnp.sqrt(jnp.sum(h * h, axis=1, keepdims=True))
    o_ref[:] = h / jnp.maximum(nrm, 1e-12)


def _mm_fin(p0, p1, h3, dega, degb, bo8):
    R = 1000
    return pl.pallas_call(
        _fin_body,
        grid=(N // R,),
        in_specs=[
            pl.BlockSpec((R, D_OUT), lambda i: (i, 0)),
            pl.BlockSpec((R, D_OUT), lambda i: (i, 0)),
            pl.BlockSpec((R, D_OUT), lambda i: (i, 0)),
            pl.BlockSpec((R, 1), lambda i: (i, 0)),
            pl.BlockSpec((R, 1), lambda i: (i, 0)),
            pl.BlockSpec((8, D_OUT), lambda i: (0, 0)),
        ],
        out_specs=pl.BlockSpec((R, D_OUT), lambda i: (i, 0)),
        out_shape=jax.ShapeDtypeStruct((N, D_OUT), jnp.float32),
    )(p0, p1, h3, dega, degb, bo8)


def kernel(x, edge_index, W_pre, b_pre, W1, b1, Wh, bh, Wo, bo):
    pad = jnp.full((E_PAD - E,), N_PAD - 1, dtype=jnp.int32)
    srcp = jnp.concatenate([edge_index[0], pad]).reshape(16, T, K)
    dstp = jnp.concatenate([edge_index[1], pad]).reshape(16, T, K)
    x_pad = jnp.zeros((N_PAD, D_IN), jnp.float32).at[:N].set(x)

    ones1 = jnp.ones((K,), jnp.float32)
    zrw = jnp.zeros((RW,), jnp.float32)
    z128 = jnp.zeros((K, 128), jnp.float32)
    b_pre8 = jnp.tile(b_pre[None, :], (8, 1))
    b18 = jnp.tile(b1[None, :], (8, 1))
    bh8 = jnp.tile(bh[None, :], (8, 1))
    bo8 = jnp.tile(bo[None, :], (8, 1))

    dega, degb = _deg_kernel(dstp, ones1, zrw)
    dega = dega[:, None]
    degb = degb[:, None]
    hs1a, hs1b = _mmA(x_pad, dega, degb, W_pre, b_pre8, W1)
    acc1a, acc1b = _conv128(hs1a, hs1b, srcp, dstp, z128)
    hs2a, hs2b = _mm_ep(acc1a, acc1b, hs1a, hs1b, dega, degb, b18,
                        Wh, D_HID, True)
    acc2a, acc2b = _conv128(hs2a, hs2b, srcp, dstp, z128)
    hs3 = _mm_ep(acc2a, acc2b, hs2a, hs2b, dega, degb, bh8, Wo, D_OUT, False)
    acc3a, acc3b = _conv_es(hs3, srcp, dstp, z128)
    return _mm_fin(acc3a, acc3b, hs3, dega, degb, bo8)


# speedup vs baseline: 18.6707x; 1.6015x over previous
import functools

import jax
import jax.numpy as jnp
from jax import lax
from jax.experimental import pallas as pl
from jax.experimental.pallas import tpu as pltpu
from jax.experimental.pallas import tpu_sc as plsc

N = 10000
E = 320000
D_IN = 128
D_FEAT = 128
D_HID = 256
D_OUT = 128

N_PAD = 10240
K = 128
T = 158
E_PAD = 16 * T * K
RW = N_PAD // 16

_mesh = plsc.VectorSubcoreMesh(core_axis_name="c", subcore_axis_name="s")


@functools.partial(
    pl.kernel,
    out_type=(jax.ShapeDtypeStruct((N_PAD,), jnp.float32),
              jax.ShapeDtypeStruct((N_PAD,), jnp.float32)),
    mesh=_mesh,
    scratch_types=[
        pltpu.VMEM_SHARED((N_PAD,), jnp.float32),
        pltpu.VMEM((K,), jnp.int32),
        pltpu.VMEM((K,), jnp.int32),
        pltpu.VMEM((K,), jnp.float32),
        pltpu.VMEM((RW,), jnp.float32),
        pltpu.SemaphoreType.DMA,
        pltpu.SemaphoreType.DMA,
    ],
    name="gcn_degree",
)
def _deg_kernel(dst_r, ones_hbm, z_hbm, out_a, out_b,
                dacc, id0, id1, ones_v, buf, si0, si1):
    cid = lax.axis_index("c")
    sid = lax.axis_index("s")
    row0 = sid * RW
    my_dst = dst_r.at[sid]
    ids = (id0, id1)
    si = (si0, si1)
    pltpu.sync_copy(ones_hbm, ones_v)
    pltpu.sync_copy(z_hbm, buf)
    pltpu.sync_copy(buf, dacc.at[pl.ds(row0, RW)])
    plsc.subcore_barrier()
    th = T // 2

    def half(t0, nch, out_c):
        pltpu.async_copy(my_dst.at[t0], ids[0], si[0])
        pltpu.async_copy(my_dst.at[t0 + 1], ids[1], si[1])

        def step(l, b):
            pltpu.make_async_copy(my_dst.at[t0], ids[b], si[b]).wait()
            pltpu.sync_copy(ones_v, dacc.at[ids[b]], add=True)

            @pl.when(l + 2 < nch)
            def _():
                pltpu.async_copy(my_dst.at[t0 + l + 2], ids[b], si[b])

        def pairb(i, c):
            step(i * 2, 0)
            step(i * 2 + 1, 1)
            return c

        lax.fori_loop(0, nch // 2, pairb, None)
        if nch % 2:
            pltpu.make_async_copy(my_dst.at[t0], ids[0], si[0]).wait()
            pltpu.sync_copy(ones_v, dacc.at[ids[0]], add=True)
        plsc.subcore_barrier()
        sl = pl.ds(row0, RW)
        pltpu.sync_copy(dacc.at[sl], buf)
        pltpu.sync_copy(buf, out_c.at[sl])

    @pl.when(cid == 0)
    def _():
        half(0, th, out_a)

    @pl.when(cid == 1)
    def _():
        half(th, T - th, out_b)


_CONV_SCRATCH = [
    pltpu.VMEM_SHARED((N_PAD, 128), jnp.float32),
    pltpu.VMEM((K,), jnp.int32),
    pltpu.VMEM((K,), jnp.int32),
    pltpu.VMEM((K,), jnp.int32),
    pltpu.VMEM((K,), jnp.int32),
    pltpu.VMEM((K, 128), jnp.float32),
    pltpu.VMEM((K, 128), jnp.float32),
    pltpu.SemaphoreType.DMA,
    pltpu.SemaphoreType.DMA,
    pltpu.SemaphoreType.DMA,
    pltpu.SemaphoreType.DMA,
]


def _zero_acc(acc, row0, z_hbm, buf):
    pltpu.sync_copy(z_hbm, buf)
    for j in range(RW // K):
        pltpu.sync_copy(buf, acc.at[pl.ds(row0 + j * K, K)])
    plsc.subcore_barrier()


def _writeout(acc, row0, out_c, buf):
    plsc.subcore_barrier()
    for j in range(RW // K):
        sl = pl.ds(row0 + j * K, K)
        pltpu.sync_copy(acc.at[sl], buf)
        pltpu.sync_copy(buf, out_c.at[sl])


def _edge_loop(hs_c, acc, my_src, my_dst, t0, nchunks,
               iss, ids, rows, sg, si):
    def wait_idx(p):
        pltpu.make_async_copy(my_src.at[t0], iss[p], si[p]).wait()
        pltpu.make_async_copy(my_src.at[t0], ids[p], si[p]).wait()

    pltpu.async_copy(my_src.at[t0], iss[0], si[0])
    pltpu.async_copy(my_dst.at[t0], ids[0], si[0])
    pltpu.async_copy(my_src.at[t0 + 1], iss[1], si[1])
    pltpu.async_copy(my_dst.at[t0 + 1], ids[1], si[1])
    wait_idx(0)
    pltpu.async_copy(hs_c.at[iss[0]], rows[0], sg[0])

    def step(l, b):
        pltpu.make_async_copy(hs_c.at[pl.ds(0, K)], rows[b], sg[b]).wait()

        @pl.when(l + 1 < nchunks)
        def _():
            wait_idx(1 - b)
            pltpu.async_copy(hs_c.at[iss[1 - b]], rows[1 - b], sg[1 - b])

        pltpu.sync_copy(rows[b], acc.at[ids[b]], add=True)

        @pl.when(l + 2 < nchunks)
        def _():
            pltpu.async_copy(my_src.at[t0 + l + 2], iss[b], si[b])
            pltpu.async_copy(my_dst.at[t0 + l + 2], ids[b], si[b])

    def pair(i, c):
        step(i * 2, 0)
        step(i * 2 + 1, 1)
        return c

    lax.fori_loop(0, nchunks // 2, pair, None)
    if nchunks % 2:
        pltpu.make_async_copy(hs_c.at[pl.ds(0, K)], rows[0], sg[0]).wait()
        pltpu.sync_copy(rows[0], acc.at[ids[0]], add=True)


@functools.partial(
    pl.kernel,
    out_type=(jax.ShapeDtypeStruct((N_PAD, 128), jnp.float32),
              jax.ShapeDtypeStruct((N_PAD, 128), jnp.float32)),
    mesh=_mesh,
    scratch_types=_CONV_SCRATCH,
    name="gcn_conv_colsplit",
)
def _conv128(hs_a, hs_b, src_r, dst_r, z_hbm, out_a, out_b,
             acc, is0, is1, id0, id1, rows0, rows1, sg0, sg1, si0, si1):
    cid = lax.axis_index("c")
    sid = lax.axis_index("s")
    row0 = sid * RW
    my_src = src_r.at[sid]
    my_dst = dst_r.at[sid]
    bufs = ((is0, is1), (id0, id1), (rows0, rows1), (sg0, sg1), (si0, si1))
    _zero_acc(acc, row0, z_hbm, rows0)

    @pl.when(cid == 0)
    def _():
        _edge_loop(hs_a, acc, my_src, my_dst, 0, T, *bufs)
        _writeout(acc, row0, out_a, rows0)

    @pl.when(cid == 1)
    def _():
        _edge_loop(hs_b, acc, my_src, my_dst, 0, T, *bufs)
        _writeout(acc, row0, out_b, rows0)


@functools.partial(
    pl.kernel,
    out_type=(jax.ShapeDtypeStruct((N_PAD, 128), jnp.float32),
              jax.ShapeDtypeStruct((N_PAD, 128), jnp.float32)),
    mesh=_mesh,
    scratch_types=_CONV_SCRATCH,
    name="gcn_conv_edgesplit",
)
def _conv_es(hs, src_r, dst_r, z_hbm, out_a, out_b,
             acc, is0, is1, id0, id1, rows0, rows1, sg0, sg1, si0, si1):
    cid = lax.axis_index("c")
    sid = lax.axis_index("s")
    row0 = sid * RW
    my_src = src_r.at[sid]
    my_dst = dst_r.at[sid]
    bufs = ((is0, is1), (id0, id1), (rows0, rows1), (sg0, sg1), (si0, si1))
    _zero_acc(acc, row0, z_hbm, rows0)
    th = T // 2

    @pl.when(cid == 0)
    def _():
        _edge_loop(hs, acc, my_src, my_dst, 0, th, *bufs)
        _writeout(acc, row0, out_a, rows0)

    @pl.when(cid == 1)
    def _():
        _edge_loop(hs, acc, my_src, my_dst, th, T - th, *bufs)
        _writeout(acc, row0, out_b, rows0)


_R = 1024
_G = N_PAD // _R


def _dinv_of(da_ref, db_ref):
    return lax.rsqrt(da_ref[:] + db_ref[:] + 1.0)


def _mmA_body(x_ref, da_ref, db_ref, wp_ref, bp_ref, w1_ref, oa_ref, ob_ref):
    dinv = _dinv_of(da_ref, db_ref)
    t = jnp.dot(x_ref[:], wp_ref[:], preferred_element_type=jnp.float32, precision=lax.Precision.HIGHEST)
    t = t + bp_ref[0:1, :]
    hs = jnp.dot(t, w1_ref[:], preferred_element_type=jnp.float32, precision=lax.Precision.HIGHEST) * dinv
    oa_ref[:] = hs[:, :128]
    ob_ref[:] = hs[:, 128:]


def _mmA(x_pad, dega, degb, W_pre, b_pre8, W1):
    return pl.pallas_call(
        _mmA_body,
        grid=(_G,),
        in_specs=[
            pl.BlockSpec((_R, D_IN), lambda i: (i, 0)),
            pl.BlockSpec((_R, 1), lambda i: (i, 0)),
            pl.BlockSpec((_R, 1), lambda i: (i, 0)),
            pl.BlockSpec((D_IN, D_FEAT), lambda i: (0, 0)),
            pl.BlockSpec((8, D_FEAT), lambda i: (0, 0)),
            pl.BlockSpec((D_FEAT, D_HID), lambda i: (0, 0)),
        ],
        out_specs=[pl.BlockSpec((_R, 128), lambda i: (i, 0))] * 2,
        out_shape=[jax.ShapeDtypeStruct((N_PAD, 128), jnp.float32)] * 2,
    )(x_pad, dega, degb, W_pre, b_pre8, W1)


def _ep_g(aa, ab, ha, hb, da_ref, db_ref, b_ref, w_ref):
    dinv = _dinv_of(da_ref, db_ref)
    g0 = jnp.maximum((aa[:] + ha[:]) * dinv + b_ref[0:1, :128], 0.0)
    g1 = jnp.maximum((ab[:] + hb[:]) * dinv + b_ref[0:1, 128:], 0.0)
    h2 = (jnp.dot(g0, w_ref[:128, :], preferred_element_type=jnp.float32, precision=lax.Precision.HIGHEST)
          + jnp.dot(g1, w_ref[128:, :], preferred_element_type=jnp.float32, precision=lax.Precision.HIGHEST))
    return h2 * dinv


def _ep_body_split(aa, ab, ha, hb, da_ref, db_ref, b_ref, w_ref, oa, ob):
    h2 = _ep_g(aa, ab, ha, hb, da_ref, db_ref, b_ref, w_ref)
    half = h2.shape[1] // 2
    oa[:] = h2[:, :half]
    ob[:] = h2[:, half:]


def _ep_body_full(aa, ab, ha, hb, da_ref, db_ref, b_ref, w_ref, o):
    o[:] = _ep_g(aa, ab, ha, hb, da_ref, db_ref, b_ref, w_ref)


def _mm_ep(acc_a, acc_b, hs_a, hs_b, dega, degb, b8, W, d_out, split):
    if split:
        body = _ep_body_split
        out_specs = [pl.BlockSpec((_R, d_out // 2), lambda i: (i, 0))] * 2
        out_shape = [jax.ShapeDtypeStruct((N_PAD, d_out // 2),
                                          jnp.float32)] * 2
    else:
        body = _ep_body_full
        out_specs = pl.BlockSpec((_R, d_out), lambda i: (i, 0))
        out_shape = jax.ShapeDtypeStruct((N_PAD, d_out), jnp.float32)
    return pl.pallas_call(
        body,
        grid=(_G,),
        in_specs=[
            pl.BlockSpec((_R, 128), lambda i: (i, 0)),
            pl.BlockSpec((_R, 128), lambda i: (i, 0)),
            pl.BlockSpec((_R, 128), lambda i: (i, 0)),
            pl.BlockSpec((_R, 128), lambda i: (i, 0)),
            pl.BlockSpec((_R, 1), lambda i: (i, 0)),
            pl.BlockSpec((_R, 1), lambda i: (i, 0)),
            pl.BlockSpec((8, D_HID), lambda i: (0, 0)),
            pl.BlockSpec((D_HID, d_out), lambda i: (0, 0)),
        ],
        out_specs=out_specs,
        out_shape=out_shape,
    )(acc_a, acc_b, hs_a, hs_b, dega, degb, b8, W)


def _fin_body(p0, p1, h3, da_ref, db_ref, b_ref, o_ref):
    dinv = _dinv_of(da_ref, db_ref)
    h = (p0[:] + p1[:] + h3[:]) * dinv + b_ref[0:1, :]
    nrm = jnp.sqrt(jnp.sum(h * h, axis=1, keepdims=True))
    o_ref[:] = h / jnp.maximum(nrm, 1e-12)


def _mm_fin(p0, p1, h3, dega, degb, bo8):
    R = 1000
    return pl.pallas_call(
        _fin_body,
        grid=(N // R,),
        in_specs=[
            pl.BlockSpec((R, D_OUT), lambda i: (i, 0)),
            pl.BlockSpec((R, D_OUT), lambda i: (i, 0)),
            pl.BlockSpec((R, D_OUT), lambda i: (i, 0)),
            pl.BlockSpec((R, 1), lambda i: (i, 0)),
            pl.BlockSpec((R, 1), lambda i: (i, 0)),
            pl.BlockSpec((8, D_OUT), lambda i: (0, 0)),
        ],
        out_specs=pl.BlockSpec((R, D_OUT), lambda i: (i, 0)),
        out_shape=jax.ShapeDtypeStruct((N, D_OUT), jnp.float32),
    )(p0, p1, h3, dega, degb, bo8)


def kernel(x, edge_index, W_pre, b_pre, W1, b1, Wh, bh, Wo, bo):
    pad = N + jnp.arange(E_PAD - E, dtype=jnp.int32) % (N_PAD - N)
    srcp = jnp.concatenate([edge_index[0], pad]).reshape(16, T, K)
    dstp = jnp.concatenate([edge_index[1], pad]).reshape(16, T, K)
    x_pad = jnp.zeros((N_PAD, D_IN), jnp.float32).at[:N].set(x)

    ones1 = jnp.ones((K,), jnp.float32)
    zrw = jnp.zeros((RW,), jnp.float32)
    z128 = jnp.zeros((K, 128), jnp.float32)
    b_pre8 = jnp.tile(b_pre[None, :], (8, 1))
    b18 = jnp.tile(b1[None, :], (8, 1))
    bh8 = jnp.tile(bh[None, :], (8, 1))
    bo8 = jnp.tile(bo[None, :], (8, 1))

    dega, degb = _deg_kernel(dstp, ones1, zrw)
    dega = dega[:, None]
    degb = degb[:, None]
    hs1a, hs1b = _mmA(x_pad, dega, degb, W_pre, b_pre8, W1)
    acc1a, acc1b = _conv128(hs1a, hs1b, srcp, dstp, z128)
    hs2a, hs2b = _mm_ep(acc1a, acc1b, hs1a, hs1b, dega, degb, b18,
                        Wh, D_HID, True)
    acc2a, acc2b = _conv128(hs2a, hs2b, srcp, dstp, z128)
    hs3 = _mm_ep(acc2a, acc2b, hs2a, hs2b, dega, degb, bh8, Wo, D_OUT, False)
    acc3a, acc3b = _conv_es(hs3, srcp, dstp, z128)
    return _mm_fin(acc3a, acc3b, hs3, dega, degb, bo8)
